# baseline (speedup 1.0000x reference)
import numpy as np
import jax
import jax.numpy as jnp
from jax.experimental import pallas as pl
from jax.experimental.pallas import tpu as pltpu

_TB = 1024
_P = 28

_KS1 = [(0, 512), (128, 640), (384, 784), (640, 784)]
_KS2 = [(0, 512), (384, 784)]


def _masks1():
    bh = np.zeros((2, 3, 28, 16), np.float32)
    bw = np.zeros((2, 3, 28, 14), np.float32)
    for p in range(2):
        for d in range(3):
            for i in range(14):
                h = 2 * i + p + d - 1
                if 0 <= h < 28:
                    bh[p, d, h, i] = 1.0
            for j in range(14):
                w = 2 * j + p + d - 1
                if 0 <= w < 28:
                    bw[p, d, w, j] = 1.0
    return bh, bw


def _masks2():
    bh = np.zeros((2, 3, 16, 8), np.float32)
    bw = np.zeros((2, 3, 14, 8), np.float32)
    for p in range(2):
        for d in range(3):
            for y in range(7):
                i = 2 * y + p + d - 1
                if 0 <= i < 14:
                    bh[p, d, i, y] = 1.0
            for x in range(7):
                j = 2 * x + p + d - 1
                if 0 <= j < 14:
                    bw[p, d, j, x] = 1.0
    return bh, bw


_BH1, _BW1 = _masks1()
_BH2, _BW2 = _masks2()

_MB1 = np.zeros((4, 1024), np.float32)
for _i in range(14):
    for _c in range(4):
        _MB1[_c, _i * 56 + _c * 14: _i * 56 + _c * 14 + 14] = 1.0
_MB2 = np.zeros((8, 512), np.float32)
for _y in range(7):
    for _o in range(8):
        _MB2[_o, _y * 64 + _o * 8: _y * 64 + _o * 8 + 7] = 1.0
_MFC = np.zeros((392, 512), np.float32)
for _o in range(8):
    for _y in range(7):
        for _x in range(7):
            _MFC[_o * 49 + _y * 7 + _x, _y * 64 + _o * 8 + _x] = 1.0


def _body(x_ref, a1_ref, a2_ref, a3_ref, a4_ref,
          b1_ref, b2_ref, b3_ref, b4_ref, o_ref):
    f32 = jnp.float32
    xb = x_ref[...].astype(jnp.bfloat16)

    hk = []
    for k in range(4):
        s0, s1 = _KS1[k]
        ck = jnp.dot(xb[:, s0:s1], a1_ref[s0:s1, 1024 * k:1024 * (k + 1)],
                     preferred_element_type=f32)
        hk.append(jnp.maximum(jnp.maximum(ck[:, 0:256], ck[:, 256:512]),
                              jnp.maximum(ck[:, 512:768], ck[:, 768:1024])))
    h1 = jnp.concatenate(hk, axis=1)
    h1 = jnp.maximum(h1 + b1_ref[...], 0.0).astype(jnp.bfloat16)

    dm = []
    for m in range(2):
        s0, s1 = _KS2[m]
        d = jnp.dot(h1[:, s0:s1], a2_ref[s0:s1, 1024 * m:1024 * (m + 1)],
                    preferred_element_type=f32)
        dm.append(jnp.maximum(jnp.maximum(d[:, 0:256], d[:, 256:512]),
                              jnp.maximum(d[:, 512:768], d[:, 768:1024])))
    h2 = jnp.concatenate(dm, axis=1)
    h2 = jnp.maximum(h2 + b2_ref[...], 0.0).astype(jnp.bfloat16)

    f = jnp.dot(h2, a3_ref[...], preferred_element_type=f32) + b3_ref[...]
    f = jnp.maximum(f, 0.0).astype(jnp.bfloat16)
    o_ref[...] = jnp.dot(f, a4_ref[...], preferred_element_type=f32) \
        + b4_ref[...]


def _stencil1(w1):
    w1r = w1.reshape(4, 3, 3).astype(jnp.float32)
    blocks = []
    for p in range(2):
        for q in range(2):
            t = jnp.einsum('cde,dHi->ceHi', w1r, _BH1[p])
            blocks.append(jnp.einsum('ceHi,eWj->HWicj', t, _BW1[q])
                          .reshape(784, 896))
    a = jnp.pad(jnp.stack(blocks), ((0, 0), (0, 0), (0, 128)))
    a = a.reshape(4, 784, 4, 256).transpose(1, 2, 0, 3)
    return a.reshape(784, 4096).astype(jnp.bfloat16)


def _stencil2(w2):
    w2r = w2.astype(jnp.float32)
    blocks = []
    for p in range(2):
        for q in range(2):
            t = jnp.einsum('ocde,dIy->oceIy', w2r, _BH2[p])
            blocks.append(jnp.einsum('oceIy,eJx->IcJyox', t, _BW2[q])
                          .reshape(896, 512))
    a = jnp.pad(jnp.stack(blocks), ((0, 0), (0, 128), (0, 0)))
    a = a.reshape(4, 1024, 2, 256).transpose(1, 2, 0, 3)
    return a.reshape(1024, 2048).astype(jnp.bfloat16)


def kernel(w1, b1, w2, b2, fc1_w, fc1_b, fc2_w, fc2_b, x):
    B = x.shape[0]
    nc = fc2_w.shape[0]
    b_pad = -(-B // _TB) * _TB
    x2 = x.reshape(B, _P * _P).astype(jnp.float32)
    if b_pad != B:
        x2 = jnp.pad(x2, ((0, b_pad - B), (0, 0)))

    f32 = jnp.float32
    a1 = _stencil1(w1)
    a2 = _stencil2(w2)
    a3 = jnp.einsum('ck,kl->lc', fc1_w.astype(f32), _MFC).astype(jnp.bfloat16)
    a4 = fc2_w.astype(f32).T.astype(jnp.bfloat16)
    b1l = (b1.astype(f32) @ _MB1).reshape(1, 1024)
    b2l = (b2.astype(f32) @ _MB2).reshape(1, 512)
    b3l = fc1_b.astype(f32).reshape(1, 32)
    b4l = fc2_b.astype(f32).reshape(1, nc)

    out = pl.pallas_call(
        _body,
        out_shape=jax.ShapeDtypeStruct((b_pad, nc), jnp.float32),
        grid=(b_pad // _TB,),
        in_specs=[
            pl.BlockSpec((_TB, _P * _P), lambda i: (i, 0)),
            pl.BlockSpec((_P * _P, 4096), lambda i: (0, 0)),
            pl.BlockSpec((1024, 2048), lambda i: (0, 0)),
            pl.BlockSpec((512, 32), lambda i: (0, 0)),
            pl.BlockSpec((32, nc), lambda i: (0, 0)),
            pl.BlockSpec((1, 1024), lambda i: (0, 0)),
            pl.BlockSpec((1, 512), lambda i: (0, 0)),
            pl.BlockSpec((1, 32), lambda i: (0, 0)),
            pl.BlockSpec((1, nc), lambda i: (0, 0)),
        ],
        out_specs=pl.BlockSpec((_TB, nc), lambda i: (i, 0)),
        compiler_params=pltpu.CompilerParams(
            dimension_semantics=("parallel",),
            vmem_limit_bytes=64 * 1024 * 1024,
        ),
    )(x2, a1, a2, a3, a4, b1l, b2l, b3l, b4l)
    return out[:B]


# speedup vs baseline: 1.9744x; 1.9744x over previous
import numpy as np
import jax
import jax.numpy as jnp
from jax.experimental import pallas as pl
from jax.experimental.pallas import tpu as pltpu

_TB = 1024
_P = 28

_KS1 = [(0, 512), (128, 640), (384, 784), (640, 784)]
_KS2 = [(0, 512), (384, 784)]


def _conv1_col(col):
    k, pq, t = col // 1024, (col % 1024) // 256, col % 256
    lane = 256 * k + t
    i, c, j = lane // 56, (lane % 56) // 14, lane % 14
    if i >= 14:
        return None
    return pq // 2, pq % 2, i, c, j


def _conv2_col(col):
    m, pq, t = col // 1024, (col % 1024) // 256, col % 256
    lane = 256 * m + t
    y, o, x = lane // 64, (lane % 64) // 8, lane % 8
    if y >= 7 or x >= 7:
        return None
    return pq // 2, pq % 2, y, o, x


def _build_statics():
    bh1 = np.zeros((3, 28, 4096), np.float32)
    bw1 = np.zeros((3, 28, 4096), np.float32)
    c1h = np.zeros((4, 4096), np.float32)
    for col in range(4096):
        dec = _conv1_col(col)
        if dec is None:
            continue
        p, q, i, c, j = dec
        c1h[c, col] = 1.0
        for d in range(3):
            h = 2 * i + p + d - 1
            if 0 <= h < 28:
                bh1[d, h, col] = 1.0
            w = 2 * j + q + d - 1
            if 0 <= w < 28:
                bw1[d, w, col] = 1.0
    bh2 = np.zeros((3, 16, 2048), np.float32)
    bw2 = np.zeros((3, 14, 2048), np.float32)
    c2h = np.zeros((8, 2048), np.float32)
    for col in range(2048):
        dec = _conv2_col(col)
        if dec is None:
            continue
        p, q, y, o, x = dec
        c2h[o, col] = 1.0
        for d in range(3):
            ii = 2 * y + p + d - 1
            if 0 <= ii < 14:
                bh2[d, ii, col] = 1.0
            jj = 2 * x + q + d - 1
            if 0 <= jj < 14:
                bw2[d, jj, col] = 1.0
    mb1 = np.zeros((4, 1024), np.float32)
    for lane in range(1024):
        i, c, j = lane // 56, (lane % 56) // 14, lane % 14
        if i < 14:
            mb1[c, lane] = 1.0
    mb2 = np.zeros((8, 512), np.float32)
    for lane in range(512):
        y, o, x = lane // 64, (lane % 64) // 8, lane % 8
        if y < 7 and x < 7:
            mb2[o, lane] = 1.0
    mfct = np.zeros((512, 392), np.float32)
    for o in range(8):
        for y in range(7):
            for x in range(7):
                mfct[y * 64 + o * 8 + x, o * 49 + y * 7 + x] = 1.0
    return bh1, bw1, c1h, bh2, bw2, c2h, mb1, mb2, mfct


(_BH1B, _BW1B, _C1H, _BH2B, _BW2B, _C2H, _MB1, _MB2, _MFCT) = _build_statics()


def _body(x_ref, w1_ref, w2_ref, b1_ref, b2_ref, fc1w_ref, fc1b_ref,
          fc2w_ref, fc2b_ref, bh1_ref, bw1_ref, c1h_ref, bh2_ref, bw2_ref,
          c2h_ref, mb1_ref, mb2_ref, mfct_ref, o_ref,
          a1_scr, a2_scr, a3_scr, a4_scr, b1_scr, b2_scr, t1_scr, t2_scr):
    f32 = jnp.float32
    bf16 = jnp.bfloat16

    @pl.when(pl.program_id(0) == 0)
    def _build():
        for e in range(3):
            acc = None
            for d in range(3):
                wcol = None
                for c in range(4):
                    term = w1_ref[c * 9 + d * 3 + e] * c1h_ref[c:c + 1, :]
                    wcol = term if wcol is None else wcol + term
                t = bh1_ref[d] * wcol
                acc = t if acc is None else acc + t
            t1_scr[e, :, :] = acc
        for hh in range(28):
            slab = None
            for e in range(3):
                s = t1_scr[e, hh:hh + 1, :] * bw1_ref[e]
                slab = s if slab is None else slab + s
            a1_scr[28 * hh:28 * hh + 28, :] = slab.astype(bf16)

        for e in range(3):
            for c in range(4):
                acc = None
                for d in range(3):
                    wcol = None
                    for o in range(8):
                        term = w2_ref[o * 36 + c * 9 + d * 3 + e] \
                            * c2h_ref[o:o + 1, :]
                        wcol = term if wcol is None else wcol + term
                    t = bh2_ref[d] * wcol
                    acc = t if acc is None else acc + t
                t2_scr[e, c * 16:c * 16 + 16, :] = acc
        for i in range(16):
            for c in range(4):
                slab = None
                for e in range(3):
                    s = t2_scr[e, c * 16 + i:c * 16 + i + 1, :] * bw2_ref[e]
                    slab = s if slab is None else slab + s
                r = i * 56 + c * 14
                a2_scr[r:r + 14, :] = slab.astype(bf16)
        a2_scr[896:1024, :] = jnp.zeros((128, 2048), bf16)

        wt = jnp.transpose(fc1w_ref[...], (1, 0)).astype(bf16)
        a3_scr[...] = jnp.dot(mfct_ref[...].astype(bf16), wt,
                              preferred_element_type=f32).astype(bf16)
        a4_scr[...] = jnp.transpose(fc2w_ref[...], (1, 0))
        b1_scr[...] = (b1_ref[0] * mb1_ref[0:1, :]
                       + b1_ref[1] * mb1_ref[1:2, :]
                       + b1_ref[2] * mb1_ref[2:3, :]
                       + b1_ref[3] * mb1_ref[3:4, :])
        acc = None
        for o in range(8):
            term = b2_ref[o] * mb2_ref[o:o + 1, :]
            acc = term if acc is None else acc + term
        b2_scr[...] = acc

    xb = x_ref[...].astype(bf16)

    hk = []
    for k in range(4):
        s0, s1 = _KS1[k]
        ck = jnp.dot(xb[:, s0:s1], a1_scr[s0:s1, 1024 * k:1024 * (k + 1)],
                     preferred_element_type=f32)
        hk.append(jnp.maximum(jnp.maximum(ck[:, 0:256], ck[:, 256:512]),
                              jnp.maximum(ck[:, 512:768], ck[:, 768:1024])))
    h1 = jnp.concatenate(hk, axis=1)
    h1 = jnp.maximum(h1 + b1_scr[...], 0.0).astype(bf16)

    dm = []
    for m in range(2):
        s0, s1 = _KS2[m]
        d = jnp.dot(h1[:, s0:s1], a2_scr[s0:s1, 1024 * m:1024 * (m + 1)],
                    preferred_element_type=f32)
        dm.append(jnp.maximum(jnp.maximum(d[:, 0:256], d[:, 256:512]),
                              jnp.maximum(d[:, 512:768], d[:, 768:1024])))
    h2 = jnp.concatenate(dm, axis=1)
    h2 = jnp.maximum(h2 + b2_scr[...], 0.0).astype(bf16)

    f = jnp.dot(h2, a3_scr[...], preferred_element_type=f32) + fc1b_ref[...]
    f = jnp.maximum(f, 0.0).astype(bf16)
    o_ref[...] = jnp.dot(f, a4_scr[...].astype(bf16),
                         preferred_element_type=f32) + fc2b_ref[...]


def kernel(w1, b1, w2, b2, fc1_w, fc1_b, fc2_w, fc2_b, x):
    B = x.shape[0]
    nc = fc2_w.shape[0]
    b_pad = -(-B // _TB) * _TB
    x2 = x.reshape(B, _P * _P).astype(jnp.float32)
    if b_pad != B:
        x2 = jnp.pad(x2, ((0, b_pad - B), (0, 0)))

    f32 = jnp.float32
    smem = pl.BlockSpec(memory_space=pltpu.MemorySpace.SMEM)
    out = pl.pallas_call(
        _body,
        out_shape=jax.ShapeDtypeStruct((b_pad, nc), jnp.float32),
        grid=(b_pad // _TB,),
        in_specs=[
            pl.BlockSpec((_TB, _P * _P), lambda i: (i, 0)),
            smem, smem, smem, smem,
            pl.BlockSpec((32, 392), lambda i: (0, 0)),
            pl.BlockSpec((1, 32), lambda i: (0, 0)),
            pl.BlockSpec((nc, 32), lambda i: (0, 0)),
            pl.BlockSpec((1, nc), lambda i: (0, 0)),
            pl.BlockSpec((3, 28, 4096), lambda i: (0, 0, 0)),
            pl.BlockSpec((3, 28, 4096), lambda i: (0, 0, 0)),
            pl.BlockSpec((4, 4096), lambda i: (0, 0)),
            pl.BlockSpec((3, 16, 2048), lambda i: (0, 0, 0)),
            pl.BlockSpec((3, 14, 2048), lambda i: (0, 0, 0)),
            pl.BlockSpec((8, 2048), lambda i: (0, 0)),
            pl.BlockSpec((4, 1024), lambda i: (0, 0)),
            pl.BlockSpec((8, 512), lambda i: (0, 0)),
            pl.BlockSpec((512, 392), lambda i: (0, 0)),
        ],
        out_specs=pl.BlockSpec((_TB, nc), lambda i: (i, 0)),
        scratch_shapes=[
            pltpu.VMEM((784, 4096), jnp.bfloat16),
            pltpu.VMEM((1024, 2048), jnp.bfloat16),
            pltpu.VMEM((512, 32), jnp.bfloat16),
            pltpu.VMEM((32, nc), jnp.float32),
            pltpu.VMEM((1, 1024), jnp.float32),
            pltpu.VMEM((1, 512), jnp.float32),
            pltpu.VMEM((3, 28, 4096), jnp.float32),
            pltpu.VMEM((3, 64, 2048), jnp.float32),
        ],
        compiler_params=pltpu.CompilerParams(
            dimension_semantics=("arbitrary",),
            vmem_limit_bytes=100 * 1024 * 1024,
        ),
    )(x2, w1.reshape(36).astype(f32), w2.reshape(288).astype(f32),
      b1.astype(f32), b2.astype(f32),
      fc1_w.astype(f32), fc1_b.astype(f32).reshape(1, 32),
      fc2_w.astype(f32), fc2_b.astype(f32).reshape(1, nc),
      _BH1B, _BW1B, _C1H, _BH2B, _BW2B, _C2H, _MB1, _MB2, _MFCT)
    return out[:B]
